# Initial kernel scaffold; baseline (speedup 1.0000x reference)
import jax
import jax.numpy as jnp
from jax.experimental import pallas as pl


def kernel(x, edge_index, W, b, gamma, beta):
    raise NotImplementedError("write your pallas kernel here")


# speedup vs baseline: 8.1497x; 8.1497x over previous
import functools

import jax
import jax.numpy as jnp
from jax import lax
from jax.experimental import pallas as pl
from jax.experimental.pallas import tpu as pltpu
from jax.experimental.pallas import tpu_sc as plsc

NC = 2
NS = 16
NW = NC * NS
CHUNK = 128

N = 10000
E = 160000
DH = 128
NPAD = N + 16
E_PAD = 163840
RW = 632
RW_LAST = N - (NS - 1) * RW
ZROWS = NPAD // NS
BLK = 1000
EPS = 1e-5

_mesh = plsc.VectorSubcoreMesh(core_axis_name="c", subcore_axis_name="s")


def _rowsplit_copy(s, srcf, dstf):
    base = s * RW

    @pl.when(s < NS - 1)
    def _():
        pltpu.sync_copy(srcf(base, RW), dstf(base, RW))

    @pl.when(s == NS - 1)
    def _():
        pltpu.sync_copy(srcf(base, RW_LAST), dstf(base, RW_LAST))


@functools.partial(
    pl.kernel,
    mesh=_mesh,
    out_type=jax.ShapeDtypeStruct((NC, N, 16), jnp.float32),
    scratch_types=[
        pltpu.VMEM_SHARED((NPAD, 16), jnp.float32),
        pltpu.VMEM((ZROWS, 16), jnp.float32),
        pltpu.VMEM((CHUNK, 16), jnp.float32),
        pltpu.VMEM((1, CHUNK), jnp.int32),
    ],
)
def _deg_kernel(dst_hbm, deg_out, hist, zbuf, ones_v, idx_v):
    c = lax.axis_index("c")
    s = lax.axis_index("s")
    tid = c * NS + s

    def fill(i, _):
        zbuf[i] = jnp.zeros((16,), jnp.float32)
        return 0

    lax.fori_loop(0, ZROWS, fill, 0, unroll=False)

    def fill1(i, _):
        ones_v[i] = jnp.ones((16,), jnp.float32)
        return 0

    lax.fori_loop(0, CHUNK, fill1, 0, unroll=False)

    pltpu.sync_copy(zbuf, hist.at[pl.ds(s * ZROWS, ZROWS)])
    plsc.subcore_barrier()

    ept = E_PAD // NW
    base = tid * ept

    def body(j, _):
        pltpu.sync_copy(dst_hbm.at[pl.ds(base + j * CHUNK, CHUNK)], idx_v.at[0])
        pltpu.sync_copy(ones_v, hist.at[idx_v.at[0]], add=True)
        return 0

    lax.fori_loop(0, ept // CHUNK, body, 0, unroll=False)
    plsc.subcore_barrier()

    _rowsplit_copy(s,
                   lambda o, n: hist.at[pl.ds(o, n)],
                   lambda o, n: deg_out.at[c, pl.ds(o, n)])


@functools.partial(
    pl.kernel,
    mesh=_mesh,
    out_type=jax.ShapeDtypeStruct((NC, N, DH), jnp.float32),
    scratch_types=[
        pltpu.VMEM_SHARED((NPAD, DH), jnp.float32),
        pltpu.VMEM((1, CHUNK), jnp.int32),
        pltpu.VMEM((1, CHUNK), jnp.int32),
        pltpu.VMEM((CHUNK, DH), jnp.float32),
        pltpu.SemaphoreType.DMA,
    ],
)
def _scatter_kernel(y0_hbm, y1_hbm, src_hbm, dst_hbm, part_out,
                    acc, sidx, didx, rows, sem):
    c = lax.axis_index("c")
    s = lax.axis_index("s")

    @pl.when(c == 0)
    def _():
        _rowsplit_copy(s,
                       lambda o, n: y0_hbm.at[pl.ds(o, n)],
                       lambda o, n: acc.at[pl.ds(o, n)])

    @pl.when(c == 1)
    def _():
        _rowsplit_copy(s,
                       lambda o, n: y1_hbm.at[pl.ds(o, n)],
                       lambda o, n: acc.at[pl.ds(o, n)])

    plsc.subcore_barrier()

    ept = E_PAD // NS
    base = s * ept

    def body(j, _):
        off = base + j * CHUNK
        pltpu.sync_copy(src_hbm.at[pl.ds(off, CHUNK)], sidx.at[0])
        pltpu.sync_copy(dst_hbm.at[pl.ds(off, CHUNK)], didx.at[0])

        @pl.when(c == 0)
        def _():
            pltpu.async_copy(y0_hbm.at[sidx.at[0]], rows, sem).wait()

        @pl.when(c == 1)
        def _():
            pltpu.async_copy(y1_hbm.at[sidx.at[0]], rows, sem).wait()

        pltpu.sync_copy(rows, acc.at[didx.at[0]], add=True)
        return 0

    lax.fori_loop(0, ept // CHUNK, body, 0, unroll=False)
    plsc.subcore_barrier()

    _rowsplit_copy(s,
                   lambda o, n: acc.at[pl.ds(o, n)],
                   lambda o, n: part_out.at[c, pl.ds(o, n)])


def _dinv_from_deg(deg_blk):
    d = deg_blk[0, :, :1] + deg_blk[1, :, :1] + 1.0
    return lax.rsqrt(d)


def _mm_body(deg_ref, x_ref, w_ref, y0_ref, y1_ref):
    dinv = _dinv_from_deg(deg_ref[...])
    xw = jnp.dot(x_ref[...], w_ref[...],
                 preferred_element_type=jnp.float32)
    y = xw * dinv
    y0_ref[...] = y[:, :DH]
    y1_ref[...] = y[:, DH:]


def _stats_body(deg_ref, part_ref, s1_ref, s2_ref):
    dinv = _dinv_from_deg(deg_ref[...])
    u = part_ref[...] * dinv[None, :, :]
    s1_ref[...] = jnp.sum(u, axis=1)[None]
    s2_ref[...] = jnp.sum(u * u, axis=1)[None]


def _bn_body(deg_ref, part_ref, s1_ref, s2_ref, b_ref, g_ref, bt_ref, o_ref):
    dinv = _dinv_from_deg(deg_ref[...])
    u = part_ref[...] * dinv[None, :, :]
    mu = jnp.sum(s1_ref[...], axis=0) / N
    var = jnp.sum(s2_ref[...], axis=0) / N - mu * mu
    b2 = b_ref[...]
    g2 = g_ref[...]
    bt2 = bt_ref[...]
    scale = g2 * lax.rsqrt(var + EPS)
    z = ((u + b2[:, None, :]) - (mu + b2)[:, None, :]) * scale[:, None, :] \
        + bt2[:, None, :]
    z = jnp.maximum(z, 0.0)
    o_ref[...] = jnp.concatenate([z[0], z[1]], axis=1)


def kernel(x, edge_index, W, b, gamma, beta):
    src = edge_index[0]
    dst = edge_index[1]
    pad = E_PAD - E
    src_p = jnp.concatenate([src, jnp.zeros((pad,), jnp.int32)])
    dst_p = jnp.concatenate([dst, jnp.full((pad,), N, jnp.int32)])

    deg_part = _deg_kernel(dst_p)

    grid = N // BLK
    y0, y1 = pl.pallas_call(
        _mm_body,
        grid=(grid,),
        in_specs=[
            pl.BlockSpec((NC, BLK, 16), lambda i: (0, i, 0)),
            pl.BlockSpec((BLK, 2 * DH), lambda i: (i, 0)),
            pl.BlockSpec((2 * DH, 2 * DH), lambda i: (0, 0)),
        ],
        out_specs=[
            pl.BlockSpec((BLK, DH), lambda i: (i, 0)),
            pl.BlockSpec((BLK, DH), lambda i: (i, 0)),
        ],
        out_shape=[
            jax.ShapeDtypeStruct((N, DH), jnp.float32),
            jax.ShapeDtypeStruct((N, DH), jnp.float32),
        ],
    )(deg_part, x, W)

    part = _scatter_kernel(y0, y1, src_p, dst_p)

    s1, s2 = pl.pallas_call(
        _stats_body,
        grid=(grid,),
        in_specs=[
            pl.BlockSpec((NC, BLK, 16), lambda i: (0, i, 0)),
            pl.BlockSpec((NC, BLK, DH), lambda i: (0, i, 0)),
        ],
        out_specs=[
            pl.BlockSpec((1, NC, DH), lambda i: (i, 0, 0)),
            pl.BlockSpec((1, NC, DH), lambda i: (i, 0, 0)),
        ],
        out_shape=[
            jax.ShapeDtypeStruct((grid, NC, DH), jnp.float32),
            jax.ShapeDtypeStruct((grid, NC, DH), jnp.float32),
        ],
    )(deg_part, part)

    out = pl.pallas_call(
        _bn_body,
        grid=(grid,),
        in_specs=[
            pl.BlockSpec((NC, BLK, 16), lambda i: (0, i, 0)),
            pl.BlockSpec((NC, BLK, DH), lambda i: (0, i, 0)),
            pl.BlockSpec((grid, NC, DH), lambda i: (0, 0, 0)),
            pl.BlockSpec((grid, NC, DH), lambda i: (0, 0, 0)),
            pl.BlockSpec((NC, DH), lambda i: (0, 0)),
            pl.BlockSpec((NC, DH), lambda i: (0, 0)),
            pl.BlockSpec((NC, DH), lambda i: (0, 0)),
        ],
        out_specs=pl.BlockSpec((BLK, 2 * DH), lambda i: (i, 0)),
        out_shape=jax.ShapeDtypeStruct((N, 2 * DH), jnp.float32),
    )(deg_part, part, s1, s2,
      b.reshape(NC, DH), gamma.reshape(NC, DH), beta.reshape(NC, DH))

    return out


# speedup vs baseline: 9.9983x; 1.2268x over previous
import functools

import jax
import jax.numpy as jnp
from jax import lax
from jax.experimental import pallas as pl
from jax.experimental.pallas import tpu as pltpu
from jax.experimental.pallas import tpu_sc as plsc

NC = 2
NS = 16
NW = NC * NS
CHUNK = 128

N = 10000
E = 160000
DH = 128
NPAD = N + 16
E_PAD = 163840
EC = E_PAD // NS // CHUNK
IB = 16
DC = E_PAD // NW // CHUNK
RW = 632
RW_LAST = N - (NS - 1) * RW
ZROWS = NPAD // NS
BLK = 1000
EPS = 1e-5

_mesh = plsc.VectorSubcoreMesh(core_axis_name="c", subcore_axis_name="s")


def _rowsplit_copy(s, srcf, dstf):
    base = s * RW

    @pl.when(s < NS - 1)
    def _():
        pltpu.sync_copy(srcf(base, RW), dstf(base, RW))

    @pl.when(s == NS - 1)
    def _():
        pltpu.sync_copy(srcf(base, RW_LAST), dstf(base, RW_LAST))


@functools.partial(
    pl.kernel,
    mesh=_mesh,
    out_type=jax.ShapeDtypeStruct((NC, N, 16), jnp.float32),
    scratch_types=[
        pltpu.VMEM_SHARED((NPAD, 16), jnp.float32),
        pltpu.VMEM((ZROWS, 16), jnp.float32),
        pltpu.VMEM((CHUNK, 16), jnp.float32),
        pltpu.VMEM((DC, CHUNK), jnp.int32),
    ],
)
def _deg_kernel(dst_hbm, deg_out, hist, zbuf, ones_v, idx_v):
    c = lax.axis_index("c")
    s = lax.axis_index("s")
    tid = c * NS + s

    def fill(i, _):
        zbuf[i] = jnp.zeros((16,), jnp.float32)
        return 0

    lax.fori_loop(0, ZROWS, fill, 0, unroll=False)

    def fill1(i, _):
        ones_v[i] = jnp.ones((16,), jnp.float32)
        return 0

    lax.fori_loop(0, CHUNK, fill1, 0, unroll=False)

    pltpu.sync_copy(dst_hbm.at[tid], idx_v)
    pltpu.sync_copy(zbuf, hist.at[pl.ds(s * ZROWS, ZROWS)])
    plsc.subcore_barrier()

    def body(j, _):
        pltpu.sync_copy(ones_v, hist.at[idx_v.at[j]], add=True)
        return 0

    lax.fori_loop(0, DC, body, 0, unroll=False)
    plsc.subcore_barrier()

    _rowsplit_copy(s,
                   lambda o, n: hist.at[pl.ds(o, n)],
                   lambda o, n: deg_out.at[c, pl.ds(o, n)])


@functools.partial(
    pl.kernel,
    mesh=_mesh,
    out_type=jax.ShapeDtypeStruct((NC, N, DH), jnp.float32),
    scratch_types=[
        pltpu.VMEM_SHARED((NPAD, DH), jnp.float32),
        pltpu.VMEM((IB, CHUNK), jnp.int32),
        pltpu.VMEM((IB, CHUNK), jnp.int32),
        pltpu.VMEM((2, CHUNK, DH), jnp.float32),
        pltpu.SemaphoreType.DMA,
    ],
)
def _scatter_kernel(y_hbm, src_hbm, dst_hbm, part_out,
                    acc, sidx, didx, rows, gsem):
    c = lax.axis_index("c")
    s = lax.axis_index("s")

    base_row = c * N
    _rowsplit_copy(s,
                   lambda o, n: y_hbm.at[pl.ds(base_row + o, n)],
                   lambda o, n: acc.at[pl.ds(o, n)])
    plsc.subcore_barrier()

    def gather(j, b):
        return pltpu.make_async_copy(y_hbm.at[sidx.at[j]], rows.at[b], gsem)

    def outer(bi, _):
        pltpu.sync_copy(src_hbm.at[c, s, pl.ds(bi * IB, IB)], sidx)
        pltpu.sync_copy(dst_hbm.at[s, pl.ds(bi * IB, IB)], didx)
        gather(0, 0).start()

        def inner(t, _):
            j0 = 2 * t
            j1 = j0 + 1
            gather(j0, 0).wait()

            @pl.when(j1 < IB)
            def _():
                gather(j1, 1).start()

            pltpu.sync_copy(rows.at[0], acc.at[didx.at[j0]], add=True)
            gather(j1, 1).wait()

            @pl.when(j1 + 1 < IB)
            def _():
                gather(j1 + 1, 0).start()

            pltpu.sync_copy(rows.at[1], acc.at[didx.at[j1]], add=True)
            return 0

        lax.fori_loop(0, IB // 2, inner, 0, unroll=False)
        return 0

    lax.fori_loop(0, EC // IB, outer, 0, unroll=False)
    plsc.subcore_barrier()

    _rowsplit_copy(s,
                   lambda o, n: acc.at[pl.ds(o, n)],
                   lambda o, n: part_out.at[c, pl.ds(o, n)])


def _dinv_from_deg(deg_blk):
    d = deg_blk[0, :, :1] + deg_blk[1, :, :1] + 1.0
    return lax.rsqrt(d)


def _mm_body(deg_ref, x_ref, w_ref, y_ref):
    dinv = _dinv_from_deg(deg_ref[...])
    xw = jnp.dot(x_ref[...], w_ref[...],
                 preferred_element_type=jnp.float32)
    y_ref[...] = (xw * dinv)[None]


def _stats_body(deg_ref, part_ref, s1_ref, s2_ref):
    dinv = _dinv_from_deg(deg_ref[...])
    u = part_ref[...] * dinv[None, :, :]
    s1_ref[...] = jnp.sum(u, axis=1)[None]
    s2_ref[...] = jnp.sum(u * u, axis=1)[None]


def _bn_body(deg_ref, part_ref, s1_ref, s2_ref, b_ref, g_ref, bt_ref, o_ref):
    dinv = _dinv_from_deg(deg_ref[...])
    u = part_ref[...] * dinv[None, :, :]
    mu = jnp.sum(s1_ref[...], axis=0) / N
    var = jnp.sum(s2_ref[...], axis=0) / N - mu * mu
    b2 = b_ref[...]
    g2 = g_ref[...]
    bt2 = bt_ref[...]
    scale = g2 * lax.rsqrt(var + EPS)
    z = ((u + b2[:, None, :]) - (mu + b2)[:, None, :]) * scale[:, None, :] \
        + bt2[:, None, :]
    z = jnp.maximum(z, 0.0)
    o_ref[...] = jnp.concatenate([z[0], z[1]], axis=1)


def kernel(x, edge_index, W, b, gamma, beta):
    src = edge_index[0]
    dst = edge_index[1]
    pad = E_PAD - E
    src_p = jnp.concatenate([src, jnp.zeros((pad,), jnp.int32)])
    dst_p = jnp.concatenate([dst, jnp.full((pad,), N, jnp.int32)])
    src2 = jnp.stack([src_p, src_p + N]).reshape(NC, NS, EC, CHUNK)
    dst_sc = dst_p.reshape(NS, EC, CHUNK)
    dst_deg = dst_p.reshape(NW, DC, CHUNK)

    deg_part = _deg_kernel(dst_deg)

    grid = N // BLK
    y = pl.pallas_call(
        _mm_body,
        grid=(grid, NC),
        in_specs=[
            pl.BlockSpec((NC, BLK, 16), lambda i, j: (0, i, 0)),
            pl.BlockSpec((BLK, 2 * DH), lambda i, j: (i, 0)),
            pl.BlockSpec((2 * DH, DH), lambda i, j: (0, j)),
        ],
        out_specs=pl.BlockSpec((1, BLK, DH), lambda i, j: (j, i, 0)),
        out_shape=jax.ShapeDtypeStruct((NC, N, DH), jnp.float32),
    )(deg_part, x, W)

    part = _scatter_kernel(y.reshape(NC * N, DH), src2, dst_sc)

    s1, s2 = pl.pallas_call(
        _stats_body,
        grid=(grid,),
        in_specs=[
            pl.BlockSpec((NC, BLK, 16), lambda i: (0, i, 0)),
            pl.BlockSpec((NC, BLK, DH), lambda i: (0, i, 0)),
        ],
        out_specs=[
            pl.BlockSpec((1, NC, DH), lambda i: (i, 0, 0)),
            pl.BlockSpec((1, NC, DH), lambda i: (i, 0, 0)),
        ],
        out_shape=[
            jax.ShapeDtypeStruct((grid, NC, DH), jnp.float32),
            jax.ShapeDtypeStruct((grid, NC, DH), jnp.float32),
        ],
    )(deg_part, part)

    out = pl.pallas_call(
        _bn_body,
        grid=(grid,),
        in_specs=[
            pl.BlockSpec((NC, BLK, 16), lambda i: (0, i, 0)),
            pl.BlockSpec((NC, BLK, DH), lambda i: (0, i, 0)),
            pl.BlockSpec((grid, NC, DH), lambda i: (0, 0, 0)),
            pl.BlockSpec((grid, NC, DH), lambda i: (0, 0, 0)),
            pl.BlockSpec((NC, DH), lambda i: (0, 0)),
            pl.BlockSpec((NC, DH), lambda i: (0, 0)),
            pl.BlockSpec((NC, DH), lambda i: (0, 0)),
        ],
        out_specs=pl.BlockSpec((BLK, 2 * DH), lambda i: (i, 0)),
        out_shape=jax.ShapeDtypeStruct((N, 2 * DH), jnp.float32),
    )(deg_part, part, s1, s2,
      b.reshape(NC, DH), gamma.reshape(NC, DH), beta.reshape(NC, DH))

    return out


# speedup vs baseline: 10.4835x; 1.0485x over previous
import functools

import jax
import jax.numpy as jnp
from jax import lax
from jax.experimental import pallas as pl
from jax.experimental.pallas import tpu as pltpu
from jax.experimental.pallas import tpu_sc as plsc

NC = 2
NS = 16
NW = NC * NS
CHUNK = 128

N = 10000
E = 160000
DH = 128
NPAD = N + 16
E_PAD = 163840
EC = E_PAD // NS // CHUNK
IB = 16
DC = E_PAD // NW // CHUNK
RW = 632
RW_LAST = N - (NS - 1) * RW
ZROWS = NPAD // NS
BLK = 1000
EPS = 1e-5

_mesh = plsc.VectorSubcoreMesh(core_axis_name="c", subcore_axis_name="s")


def _rowsplit_copy(s, srcf, dstf):
    base = s * RW

    @pl.when(s < NS - 1)
    def _():
        pltpu.sync_copy(srcf(base, RW), dstf(base, RW))

    @pl.when(s == NS - 1)
    def _():
        pltpu.sync_copy(srcf(base, RW_LAST), dstf(base, RW_LAST))


@functools.partial(
    pl.kernel,
    mesh=_mesh,
    out_type=jax.ShapeDtypeStruct((NC, N, 16), jnp.float32),
    scratch_types=[
        pltpu.VMEM_SHARED((NPAD, 16), jnp.float32),
        pltpu.VMEM((ZROWS, 16), jnp.float32),
        pltpu.VMEM((CHUNK, 16), jnp.float32),
        pltpu.VMEM((DC, CHUNK), jnp.int32),
        pltpu.SemaphoreType.DMA,
    ],
)
def _deg_kernel(dst_hbm, deg_out, hist, zbuf, ones_v, idx_v, dsem):
    c = lax.axis_index("c")
    s = lax.axis_index("s")
    tid = c * NS + s

    def fill(i, _):
        zbuf[i] = jnp.zeros((16,), jnp.float32)
        return 0

    lax.fori_loop(0, ZROWS, fill, 0, unroll=False)

    def fill1(i, _):
        ones_v[i] = jnp.ones((16,), jnp.float32)
        return 0

    lax.fori_loop(0, CHUNK, fill1, 0, unroll=False)

    pltpu.sync_copy(dst_hbm.at[tid], idx_v)
    pltpu.sync_copy(zbuf, hist.at[pl.ds(s * ZROWS, ZROWS)])
    plsc.subcore_barrier()

    def body(j, _):
        pltpu.sync_copy(ones_v, hist.at[idx_v.at[j]], add=True)
        return 0

    lax.fori_loop(0, DC, body, 0, unroll=False)
    plsc.subcore_barrier()

    _rowsplit_copy(s,
                   lambda o, n: hist.at[pl.ds(o, n)],
                   lambda o, n: deg_out.at[c, pl.ds(o, n)])


@functools.partial(
    pl.kernel,
    mesh=_mesh,
    out_type=jax.ShapeDtypeStruct((NC, N, DH), jnp.float32),
    scratch_types=[
        pltpu.VMEM_SHARED((NPAD, DH), jnp.float32),
        pltpu.VMEM((IB, CHUNK), jnp.int32),
        pltpu.VMEM((IB, CHUNK), jnp.int32),
        pltpu.VMEM((2, CHUNK, DH), jnp.float32),
        pltpu.SemaphoreType.DMA,
        pltpu.SemaphoreType.DMA,
    ],
)
def _scatter_kernel(y_hbm, src_hbm, dst_hbm, part_out,
                    acc, sidx, didx, rows, gsem0, gsem1):
    c = lax.axis_index("c")
    s = lax.axis_index("s")

    base_row = c * N
    _rowsplit_copy(s,
                   lambda o, n: y_hbm.at[pl.ds(base_row + o, n)],
                   lambda o, n: acc.at[pl.ds(o, n)])
    plsc.subcore_barrier()

    def gather(j, b, sem):
        return pltpu.make_async_copy(y_hbm.at[sidx.at[j]], rows.at[b], sem)

    def outer(bi, _):
        pltpu.sync_copy(src_hbm.at[c, s, pl.ds(bi * IB, IB)], sidx)
        pltpu.sync_copy(dst_hbm.at[s, pl.ds(bi * IB, IB)], didx)
        gather(0, 0, gsem0).start()
        gather(1, 1, gsem1).start()

        def inner(t, _):
            j0 = 2 * t
            j1 = j0 + 1
            gather(j0, 0, gsem0).wait()
            pltpu.sync_copy(rows.at[0], acc.at[didx.at[j0]], add=True)

            @pl.when(j0 + 2 < IB)
            def _():
                gather(j0 + 2, 0, gsem0).start()

            gather(j1, 1, gsem1).wait()
            pltpu.sync_copy(rows.at[1], acc.at[didx.at[j1]], add=True)

            @pl.when(j1 + 2 < IB)
            def _():
                gather(j1 + 2, 1, gsem1).start()

            return 0

        lax.fori_loop(0, IB // 2, inner, 0, unroll=False)
        return 0

    lax.fori_loop(0, EC // IB, outer, 0, unroll=False)
    plsc.subcore_barrier()

    _rowsplit_copy(s,
                   lambda o, n: acc.at[pl.ds(o, n)],
                   lambda o, n: part_out.at[c, pl.ds(o, n)])


def _dinv_from_deg(deg_blk):
    d = deg_blk[0, :, :1] + deg_blk[1, :, :1] + 1.0
    return lax.rsqrt(d)


def _mm_body(deg_ref, x_ref, w_ref, y_ref):
    dinv = _dinv_from_deg(deg_ref[...])
    xw = jnp.dot(x_ref[...], w_ref[...],
                 preferred_element_type=jnp.float32)
    y_ref[...] = (xw * dinv)[None]


def _stats_body(deg_ref, part_ref, s1_ref, s2_ref):
    dinv = _dinv_from_deg(deg_ref[...])
    u = part_ref[...] * dinv[None, :, :]
    s1_ref[...] = jnp.sum(u, axis=1)[None]
    s2_ref[...] = jnp.sum(u * u, axis=1)[None]


def _bn_body(deg_ref, part_ref, s1_ref, s2_ref, b_ref, g_ref, bt_ref, o_ref):
    dinv = _dinv_from_deg(deg_ref[...])
    u = part_ref[...] * dinv[None, :, :]
    mu = jnp.sum(s1_ref[...], axis=0) / N
    var = jnp.sum(s2_ref[...], axis=0) / N - mu * mu
    b2 = b_ref[...]
    g2 = g_ref[...]
    bt2 = bt_ref[...]
    scale = g2 * lax.rsqrt(var + EPS)
    z = ((u + b2[:, None, :]) - (mu + b2)[:, None, :]) * scale[:, None, :] \
        + bt2[:, None, :]
    z = jnp.maximum(z, 0.0)
    o_ref[...] = jnp.concatenate([z[0], z[1]], axis=1)


def kernel(x, edge_index, W, b, gamma, beta):
    src = edge_index[0]
    dst = edge_index[1]
    pad = E_PAD - E
    src_p = jnp.concatenate([src, jnp.zeros((pad,), jnp.int32)])
    dst_p = jnp.concatenate([dst, jnp.full((pad,), N, jnp.int32)])
    src2 = jnp.stack([src_p, src_p + N]).reshape(NC, NS, EC, CHUNK)
    dst_sc = dst_p.reshape(NS, EC, CHUNK)
    dst_deg = dst_p.reshape(NW, DC, CHUNK)

    deg_part = _deg_kernel(dst_deg)

    grid = N // BLK
    y = pl.pallas_call(
        _mm_body,
        grid=(grid, NC),
        in_specs=[
            pl.BlockSpec((NC, BLK, 16), lambda i, j: (0, i, 0)),
            pl.BlockSpec((BLK, 2 * DH), lambda i, j: (i, 0)),
            pl.BlockSpec((2 * DH, DH), lambda i, j: (0, j)),
        ],
        out_specs=pl.BlockSpec((1, BLK, DH), lambda i, j: (j, i, 0)),
        out_shape=jax.ShapeDtypeStruct((NC, N, DH), jnp.float32),
    )(deg_part, x, W)

    part = _scatter_kernel(y.reshape(NC * N, DH), src2, dst_sc)

    s1, s2 = pl.pallas_call(
        _stats_body,
        grid=(grid,),
        in_specs=[
            pl.BlockSpec((NC, BLK, 16), lambda i: (0, i, 0)),
            pl.BlockSpec((NC, BLK, DH), lambda i: (0, i, 0)),
        ],
        out_specs=[
            pl.BlockSpec((1, NC, DH), lambda i: (i, 0, 0)),
            pl.BlockSpec((1, NC, DH), lambda i: (i, 0, 0)),
        ],
        out_shape=[
            jax.ShapeDtypeStruct((grid, NC, DH), jnp.float32),
            jax.ShapeDtypeStruct((grid, NC, DH), jnp.float32),
        ],
    )(deg_part, part)

    out = pl.pallas_call(
        _bn_body,
        grid=(grid,),
        in_specs=[
            pl.BlockSpec((NC, BLK, 16), lambda i: (0, i, 0)),
            pl.BlockSpec((NC, BLK, DH), lambda i: (0, i, 0)),
            pl.BlockSpec((grid, NC, DH), lambda i: (0, 0, 0)),
            pl.BlockSpec((grid, NC, DH), lambda i: (0, 0, 0)),
            pl.BlockSpec((NC, DH), lambda i: (0, 0)),
            pl.BlockSpec((NC, DH), lambda i: (0, 0)),
            pl.BlockSpec((NC, DH), lambda i: (0, 0)),
        ],
        out_specs=pl.BlockSpec((BLK, 2 * DH), lambda i: (i, 0)),
        out_shape=jax.ShapeDtypeStruct((N, 2 * DH), jnp.float32),
    )(deg_part, part, s1, s2,
      b.reshape(NC, DH), gamma.reshape(NC, DH), beta.reshape(NC, DH))

    return out
